# Initial kernel scaffold; baseline (speedup 1.0000x reference)
import jax
import jax.numpy as jnp
from jax.experimental import pallas as pl


def kernel(x, W, b):
    raise NotImplementedError("write your pallas kernel here")


# speedup vs baseline: 114.0562x; 114.0562x over previous
import functools

import jax
import jax.numpy as jnp
from jax import lax
from jax.experimental import pallas as pl
from jax.experimental.pallas import tpu as pltpu
from jax.experimental.pallas import tpu_sc as plsc

BATCH = 1024
HIST = 20
VOCAB = 100000

NUM_CORES = 2
NUM_SUBCORES = 16
NUM_TILES = NUM_CORES * NUM_SUBCORES
ROWS_PER_TILE = BATCH // NUM_TILES
GROUPS = ROWS_PER_TILE // 16


def _sc_bow_kernel(x_hbm, w_hbm, b_hbm, out_hbm, idx_v, vals_v, b_v, out_v,
                   sem):
    c = lax.axis_index("c")
    s = lax.axis_index("s")
    wid = s * NUM_CORES + c

    pltpu.sync_copy(x_hbm.at[wid], idx_v)
    pltpu.sync_copy(b_hbm, b_v)

    copies = [
        pltpu.async_copy(w_hbm.at[idx_v.at[l]], vals_v.at[l], sem)
        for l in range(HIST)
    ]
    for cp in copies:
        cp.wait()

    bias = b_v[...]
    for g in range(GROUPS):
        sl = pl.ds(g * 16, 16)
        idx = [idx_v[l, sl] for l in range(HIST)]
        acc = bias
        for l in range(HIST):
            dup = None
            for j in range(l):
                eq = idx[j] == idx[l]
                dup = eq if dup is None else (dup | eq)
            val = vals_v[l, sl]
            if dup is not None:
                val = jnp.where(dup, 0.0, val)
            acc = acc + val
        out_v[sl] = acc

    pltpu.sync_copy(out_v, out_hbm.at[pl.ds(wid * ROWS_PER_TILE,
                                            ROWS_PER_TILE)])


@jax.jit
def _bow_forward(x, W, b):
    xh = x.reshape(NUM_TILES, ROWS_PER_TILE, HIST).transpose(0, 2, 1)
    w_flat = W.reshape(VOCAB)
    b_vec = jnp.broadcast_to(b, (16,))

    mesh = plsc.VectorSubcoreMesh(core_axis_name="c", subcore_axis_name="s")
    run = pl.kernel(
        _sc_bow_kernel,
        mesh=mesh,
        out_type=jax.ShapeDtypeStruct((BATCH,), jnp.float32),
        scratch_types=[
            pltpu.VMEM((HIST, ROWS_PER_TILE), jnp.int32),
            pltpu.VMEM((HIST, ROWS_PER_TILE), jnp.float32),
            pltpu.VMEM((16,), jnp.float32),
            pltpu.VMEM((ROWS_PER_TILE,), jnp.float32),
            pltpu.SemaphoreType.DMA,
        ],
    )
    out_flat = run(xh, w_flat, b_vec)
    return out_flat.reshape(BATCH, 1)


def kernel(x, W, b):
    return _bow_forward(x, W, b)


# speedup vs baseline: 115.6357x; 1.0138x over previous
import functools

import jax
import jax.numpy as jnp
from jax import lax
from jax.experimental import pallas as pl
from jax.experimental.pallas import tpu as pltpu
from jax.experimental.pallas import tpu_sc as plsc

BATCH = 1024
HIST = 20
VOCAB = 100000

NUM_CORES = 2
NUM_SUBCORES = 16
NUM_TILES = NUM_CORES * NUM_SUBCORES
ROWS_PER_TILE = BATCH // NUM_TILES
GROUPS = ROWS_PER_TILE // 16
CHUNK = 128
CHUNKS = HIST * ROWS_PER_TILE // CHUNK


def _sc_bow_kernel(x_hbm, w_hbm, b_hbm, out_hbm, idx_v, vals_v, b_v, out_v,
                   sem):
    c = lax.axis_index("c")
    s = lax.axis_index("s")
    wid = s * NUM_CORES + c

    pltpu.sync_copy(x_hbm.at[wid], idx_v)
    pltpu.sync_copy(b_hbm, b_v)

    copies = [
        pltpu.async_copy(w_hbm.at[idx_v.at[p]], vals_v.at[p], sem)
        for p in range(CHUNKS)
    ]
    for cp in copies:
        cp.wait()

    def flat(l, g):
        o = l * ROWS_PER_TILE + g * 16
        return o // CHUNK, pl.ds(o % CHUNK, 16)

    bias = b_v[...]
    for g in range(GROUPS):
        idx = []
        for l in range(HIST):
            p, sl = flat(l, g)
            idx.append(idx_v[p, sl])
        acc = bias
        for l in range(HIST):
            dup = None
            for j in range(l):
                eq = idx[j] == idx[l]
                dup = eq if dup is None else (dup | eq)
            p, sl = flat(l, g)
            val = vals_v[p, sl]
            if dup is not None:
                val = jnp.where(dup, 0.0, val)
            acc = acc + val
        out_v[pl.ds(g * 16, 16)] = acc

    pltpu.sync_copy(out_v, out_hbm.at[pl.ds(wid * ROWS_PER_TILE,
                                            ROWS_PER_TILE)])


@jax.jit
def _bow_forward(x, W, b):
    xh = (x.reshape(NUM_TILES, ROWS_PER_TILE, HIST)
          .transpose(0, 2, 1).reshape(NUM_TILES, CHUNKS, CHUNK))
    w_flat = W.reshape(VOCAB)
    b_vec = jnp.broadcast_to(b, (16,))

    mesh = plsc.VectorSubcoreMesh(core_axis_name="c", subcore_axis_name="s")
    run = pl.kernel(
        _sc_bow_kernel,
        mesh=mesh,
        out_type=jax.ShapeDtypeStruct((BATCH,), jnp.float32),
        scratch_types=[
            pltpu.VMEM((CHUNKS, CHUNK), jnp.int32),
            pltpu.VMEM((CHUNKS, CHUNK), jnp.float32),
            pltpu.VMEM((16,), jnp.float32),
            pltpu.VMEM((ROWS_PER_TILE,), jnp.float32),
            pltpu.SemaphoreType.DMA,
        ],
    )
    out_flat = run(xh, w_flat, b_vec)
    return out_flat.reshape(BATCH, 1)


def kernel(x, W, b):
    return _bow_forward(x, W, b)
